# Initial kernel scaffold; baseline (speedup 1.0000x reference)
import jax
import jax.numpy as jnp
from jax.experimental import pallas as pl


def kernel(X, word_table, pos_table):
    raise NotImplementedError("write your pallas kernel here")


# speedup vs baseline: 7.5003x; 7.5003x over previous
import functools

import jax
import jax.numpy as jnp
from jax import lax
from jax.experimental import pallas as pl

_B = 16384
_P = 12
_V = 28
_D = 24
_BB = 2048


def _tc_body(x_ref, word_ref, pos_ref, out_ref):
    x = x_ref[...]
    w = word_ref[...]
    iota_v = lax.broadcasted_iota(jnp.int32, (1, _V), 1)
    for p in range(_P):
        xp = x[:, p:p + 1]
        oh = (xp == iota_v).astype(jnp.float32)
        res = lax.dot_general(
            oh, w, dimension_numbers=(((1,), (0,)), ((), ())),
            preferred_element_type=jnp.float32)
        out_ref[:, p, :] = res + pos_ref[pl.ds(p, 1), :]


def kernel(X, word_table, pos_table):
    grid = (_B // _BB,)
    return pl.pallas_call(
        _tc_body,
        grid=grid,
        in_specs=[
            pl.BlockSpec((_BB, _P), lambda i: (i, 0)),
            pl.BlockSpec((_V, _D), lambda i: (0, 0)),
            pl.BlockSpec((_P, _D), lambda i: (0, 0)),
        ],
        out_specs=pl.BlockSpec((_BB, _P, _D), lambda i: (i, 0, 0)),
        out_shape=jax.ShapeDtypeStruct((_B, _P, _D), jnp.float32),
    )(X.astype(jnp.int32), word_table, pos_table)


# speedup vs baseline: 7.6610x; 1.0214x over previous
import jax
import jax.numpy as jnp
from jax import lax
from jax.experimental import pallas as pl

_B = 16384
_P = 12
_V = 28
_D = 24
_BB = 1024


def _tc_body(x_ref, word_ref, pos_ref, out_ref):
    x = x_ref[...]
    w = word_ref[...]
    iota_v = lax.broadcasted_iota(jnp.int32, (1, _V), 1)
    for p in range(_P):
        cp = w + pos_ref[pl.ds(p, 1), :]
        xp = x[:, p:p + 1]
        oh = (xp == iota_v).astype(jnp.float32)
        out_ref[:, p, :] = lax.dot_general(
            oh, cp, dimension_numbers=(((1,), (0,)), ((), ())),
            preferred_element_type=jnp.float32)


def kernel(X, word_table, pos_table):
    grid = (_B // _BB,)
    return pl.pallas_call(
        _tc_body,
        grid=grid,
        in_specs=[
            pl.BlockSpec((_BB, _P), lambda i: (i, 0)),
            pl.BlockSpec((_V, _D), lambda i: (0, 0)),
            pl.BlockSpec((_P, _D), lambda i: (0, 0)),
        ],
        out_specs=pl.BlockSpec((_BB, _P, _D), lambda i: (i, 0, 0)),
        out_shape=jax.ShapeDtypeStruct((_B, _P, _D), jnp.float32),
    )(X.astype(jnp.int32), word_table, pos_table)


# speedup vs baseline: 7.6651x; 1.0005x over previous
import jax
import jax.numpy as jnp
from jax import lax
from jax.experimental import pallas as pl

_B = 16384
_P = 12
_V = 28
_D = 24
_BB = 512


def _tc_body(x_ref, word_ref, pos_ref, out_ref):
    x = x_ref[...]
    w = word_ref[...]
    iota_v = lax.broadcasted_iota(jnp.int32, (1, _V), 1)
    for p in range(_P):
        cp = w + pos_ref[pl.ds(p, 1), :]
        xp = x[:, p:p + 1]
        oh = (xp == iota_v).astype(jnp.float32)
        out_ref[:, p, :] = lax.dot_general(
            oh, cp, dimension_numbers=(((1,), (0,)), ((), ())),
            preferred_element_type=jnp.float32)


def kernel(X, word_table, pos_table):
    grid = (_B // _BB,)
    return pl.pallas_call(
        _tc_body,
        grid=grid,
        in_specs=[
            pl.BlockSpec((_BB, _P), lambda i: (i, 0)),
            pl.BlockSpec((_V, _D), lambda i: (0, 0)),
            pl.BlockSpec((_P, _D), lambda i: (0, 0)),
        ],
        out_specs=pl.BlockSpec((_BB, _P, _D), lambda i: (i, 0, 0)),
        out_shape=jax.ShapeDtypeStruct((_B, _P, _D), jnp.float32),
    )(X.astype(jnp.int32), word_table, pos_table)
